# baseline (speedup 1.0000x reference)
import jax
import jax.numpy as jnp
from jax.experimental import pallas as pl
from jax.experimental.pallas import tpu as pltpu


def _matmul_bf16_kernel(x_ref, ae_ref, o_ref):
    o_ref[...] = jnp.dot(
        x_ref[...].astype(jnp.bfloat16),
        ae_ref[...],
        preferred_element_type=jnp.float32,
    ).astype(o_ref.dtype)


@jax.jit
def kernel(x, A):
    N, C, V, L = x.shape
    V2, W = A.shape
    assert V == V2
    B = N * C
    K = V * L
    Nw = W * L
    x2 = x.reshape(B, K)
    ae = jnp.kron(A.astype(jnp.bfloat16), jnp.eye(L, dtype=jnp.bfloat16))

    tb = min(128, B)
    grid = pl.cdiv(B, tb)
    footprint = K * Nw * 2 + 2 * tb * K * 4 + 2 * tb * Nw * 4
    vmem_limit = int(min(56 << 20, max(16 << 20, footprint + (4 << 20))))

    out = pl.pallas_call(
        _matmul_bf16_kernel,
        out_shape=jax.ShapeDtypeStruct((B, Nw), x.dtype),
        grid=(grid,),
        in_specs=[
            pl.BlockSpec((tb, K), lambda i: (i, 0)),
            pl.BlockSpec((K, Nw), lambda i: (0, 0)),
        ],
        out_specs=pl.BlockSpec((tb, Nw), lambda i: (i, 0)),
        compiler_params=pltpu.CompilerParams(
            dimension_semantics=("parallel",),
            vmem_limit_bytes=vmem_limit,
        ),
    )(x2, ae)
    return out.reshape(N, C, W, L)


# speedup vs baseline: 1.8141x; 1.8141x over previous
import functools

import jax
import jax.numpy as jnp
from jax.experimental import pallas as pl
from jax.experimental.pallas import tpu as pltpu


def _nconv_kron_kernel(x_ref, a_ref, o_ref, ae_ref, *, V, W, L, TR):
    K = V * L
    Nw = W * L

    @pl.when(pl.program_id(1) == 0)
    def _build_kron():
        a = a_ref[...].astype(jnp.bfloat16)
        ar = jnp.broadcast_to(a[:, None, :], (V, L, W)).reshape(K, W)
        col = jax.lax.broadcasted_iota(jnp.int32, (W, Nw), 1)
        row = jax.lax.broadcasted_iota(jnp.int32, (W, Nw), 0)
        rep = (col // L == row).astype(jnp.bfloat16)
        mrow = jax.lax.broadcasted_iota(jnp.int32, (TR, Nw), 0)
        mcol = jax.lax.broadcasted_iota(jnp.int32, (TR, Nw), 1)
        mask = ((mrow % L) == (mcol % L)).astype(jnp.bfloat16)
        for t in range(K // TR):
            arl = jnp.dot(ar[t * TR:(t + 1) * TR, :], rep,
                          preferred_element_type=jnp.float32)
            ae_ref[t * TR:(t + 1) * TR, :] = arl.astype(jnp.bfloat16) * mask

    o_ref[...] = jnp.dot(
        x_ref[...].astype(jnp.bfloat16),
        ae_ref[...],
        preferred_element_type=jnp.float32,
    ).astype(o_ref.dtype)


@jax.jit
def kernel(x, A):
    N, C, V, L = x.shape
    V2, W = A.shape
    assert V == V2
    B = N * C
    K = V * L
    Nw = W * L
    x2 = x.reshape(B, K)

    tb = min(128, B)
    nblk = pl.cdiv(B, tb)
    ncores = 2 if nblk % 2 == 0 else 1
    g2 = nblk // ncores
    TR = min(256, K)
    assert K % TR == 0 and TR % L == 0

    footprint = (K * Nw * 2
                 + 2 * tb * K * 4
                 + 2 * tb * Nw * 4
                 + W * Nw * 2
                 + 2 * TR * Nw * 2)
    vmem_limit = int(min(56 << 20, footprint + (8 << 20)))

    out = pl.pallas_call(
        functools.partial(_nconv_kron_kernel, V=V, W=W, L=L, TR=TR),
        out_shape=jax.ShapeDtypeStruct((B, Nw), x.dtype),
        grid=(ncores, g2),
        in_specs=[
            pl.BlockSpec((tb, K), lambda i, j, g2=g2: (i * g2 + j, 0)),
            pl.BlockSpec((V, W), lambda i, j: (0, 0)),
        ],
        out_specs=pl.BlockSpec((tb, Nw), lambda i, j, g2=g2: (i * g2 + j, 0)),
        scratch_shapes=[pltpu.VMEM((K, Nw), jnp.bfloat16)],
        compiler_params=pltpu.CompilerParams(
            dimension_semantics=("parallel", "arbitrary"),
            vmem_limit_bytes=vmem_limit,
        ),
    )(x2, A)
    return out.reshape(N, C, W, L)


# speedup vs baseline: 1.8152x; 1.0006x over previous
import functools

import jax
import jax.numpy as jnp
from jax.experimental import pallas as pl
from jax.experimental.pallas import tpu as pltpu


def _nconv_kron_kernel(x_ref, a_ref, o_ref, ae_ref, *, V, W, L, TR):
    K = V * L
    Nw = W * L

    @pl.when(pl.program_id(1) == 0)
    def _build_kron():
        a = a_ref[...].astype(jnp.bfloat16)
        ar = jnp.broadcast_to(a[:, None, :], (V, L, W)).reshape(K, W)
        col = jax.lax.broadcasted_iota(jnp.int32, (W, Nw), 1)
        row = jax.lax.broadcasted_iota(jnp.int32, (W, Nw), 0)
        rep = (col // L == row).astype(jnp.bfloat16)
        mrow = jax.lax.broadcasted_iota(jnp.int32, (TR, Nw), 0)
        mcol = jax.lax.broadcasted_iota(jnp.int32, (TR, Nw), 1)
        mask = ((mrow % L) == (mcol % L)).astype(jnp.bfloat16)
        for t in range(K // TR):
            arl = jnp.dot(ar[t * TR:(t + 1) * TR, :], rep,
                          preferred_element_type=jnp.float32)
            ae_ref[t * TR:(t + 1) * TR, :] = arl.astype(jnp.bfloat16) * mask

    o_ref[...] = jnp.dot(
        x_ref[...].astype(jnp.bfloat16),
        ae_ref[...],
        preferred_element_type=jnp.float32,
        precision=jax.lax.Precision.DEFAULT,
    ).astype(o_ref.dtype)


@jax.jit
def kernel(x, A):
    N, C, V, L = x.shape
    V2, W = A.shape
    assert V == V2
    B = N * C
    K = V * L
    Nw = W * L
    x2 = x.reshape(B, K)

    tb = min(128, B)
    nblk = pl.cdiv(B, tb)
    ncores = 2 if nblk % 2 == 0 else 1
    g2 = nblk // ncores
    TR = min(256, K)
    assert K % TR == 0 and TR % L == 0

    footprint = (K * Nw * 2
                 + 2 * tb * K * 4
                 + 2 * tb * Nw * 4
                 + W * Nw * 2
                 + 2 * TR * Nw * 2)
    vmem_limit = int(min(56 << 20, footprint + (8 << 20)))

    out = pl.pallas_call(
        functools.partial(_nconv_kron_kernel, V=V, W=W, L=L, TR=TR),
        out_shape=jax.ShapeDtypeStruct((B, Nw), x.dtype),
        grid=(ncores, g2),
        in_specs=[
            pl.BlockSpec((tb, K), lambda i, j, g2=g2: (i * g2 + j, 0)),
            pl.BlockSpec((V, W), lambda i, j: (0, 0)),
        ],
        out_specs=pl.BlockSpec((tb, Nw), lambda i, j, g2=g2: (i * g2 + j, 0)),
        scratch_shapes=[pltpu.VMEM((K, Nw), jnp.bfloat16)],
        compiler_params=pltpu.CompilerParams(
            dimension_semantics=("parallel", "arbitrary"),
            vmem_limit_bytes=vmem_limit,
        ),
    )(x2, A)
    return out.reshape(N, C, W, L)


# speedup vs baseline: 12.4642x; 6.8664x over previous
import jax
import jax.numpy as jnp
from jax.experimental import pallas as pl
from jax.experimental.pallas import tpu as pltpu


def _matmul_kernel(at_ref, x_ref, o_ref):
    o_ref[...] = jnp.dot(
        at_ref[...],
        x_ref[...],
        preferred_element_type=jnp.float32,
        precision=jax.lax.Precision.DEFAULT,
    ).astype(o_ref.dtype)


@jax.jit
def kernel(x, A):
    N, C, V, L = x.shape
    V2, W = A.shape
    assert V == V2
    M = N * C * L

    x_t = jnp.transpose(x.astype(jnp.bfloat16), (2, 0, 1, 3)).reshape(V, M)
    a_t = jnp.transpose(A).astype(jnp.bfloat16)

    tl = min(2048, M)
    grid = pl.cdiv(M, tl)

    out_t = pl.pallas_call(
        _matmul_kernel,
        out_shape=jax.ShapeDtypeStruct((W, M), jnp.bfloat16),
        grid=(grid,),
        in_specs=[
            pl.BlockSpec((W, V), lambda j: (0, 0)),
            pl.BlockSpec((V, tl), lambda j: (0, j)),
        ],
        out_specs=pl.BlockSpec((W, tl), lambda j: (0, j)),
        compiler_params=pltpu.CompilerParams(
            dimension_semantics=("parallel",),
            vmem_limit_bytes=int(32 << 20),
        ),
    )(a_t, x_t)

    out = out_t.reshape(W, N, C, L).transpose(1, 2, 0, 3).astype(jnp.float32)
    return out
